# baseline (speedup 1.0000x reference)
import functools

import jax
import jax.numpy as jnp
from jax import lax
from jax.experimental import pallas as pl
from jax.experimental.pallas import tpu as pltpu
from jax.experimental.pallas import tpu_sc as plsc

_P = 8
_DP = 96
_AD = 8
_ADP = 128
_H = 64
_LD = _P * _DP
_BLK = 512
_NW = 32
_CH = 128


def _routing_metadata(pol, n_pad, n_blk):
    n = pol.shape[0]
    oh = (pol[:, None] == jnp.arange(_P, dtype=jnp.int32)[None, :]).astype(jnp.int32)
    rank = jnp.take_along_axis(jnp.cumsum(oh, axis=0) - oh, pol[:, None], axis=1)[:, 0]
    counts = jnp.sum(oh, axis=0)
    padded = ((counts + _BLK - 1) // _BLK) * _BLK
    seg_end = jnp.cumsum(padded)
    seg_start = seg_end - padded
    slot = seg_start[pol] + rank
    gidx = jnp.zeros((n_pad,), jnp.int32).at[slot].set(
        jnp.arange(n, dtype=jnp.int32))
    bpol = jnp.searchsorted(
        seg_end, jnp.arange(n_blk, dtype=jnp.int32) * _BLK, side="right")
    bpol = jnp.minimum(bpol, _P - 1).astype(jnp.int32)
    return slot, gidx, bpol


def _pack_weights(W1, b1, W2, b2):
    Wz = W1[:, :, :_DP, :]
    W1z = jnp.zeros((_P, 4, 2 * _DP, 2 * _H), W1.dtype)
    W1z = W1z.at[:, :, :_DP, :_H].set(Wz[:, 0::2])
    W1z = W1z.at[:, :, _DP:, _H:].set(Wz[:, 1::2])
    Wa = jnp.transpose(W1[:, :, _DP:, :], (0, 2, 1, 3)).reshape(_P, _AD, _P * _H)
    W1a = jnp.zeros((_P, _ADP, _P * _H), W1.dtype).at[:, :_AD, :].set(Wa)
    b1f = b1.reshape(_P, 1, _P * _H)
    W2p = jnp.zeros((_P, 4, 2 * _H, 2 * _DP), W2.dtype)
    W2p = W2p.at[:, :, :_H, :_DP].set(W2[:, 0::2])
    W2p = W2p.at[:, :, _H:, _DP:].set(W2[:, 1::2])
    b2f = b2.reshape(_P, 1, _P * _DP)
    return W1z, W1a, b1f, W2p, b2f


def _sc_mesh():
    return plsc.VectorSubcoreMesh(core_axis_name="c", subcore_axis_name="s")


def _gather_sorted(latents, actions_p, gidx, n_pad):
    rows_per_w = n_pad // _NW
    n_ch = rows_per_w // _CH

    @functools.partial(
        pl.kernel,
        out_type=(
            jax.ShapeDtypeStruct((n_pad, _LD), jnp.float32),
            jax.ShapeDtypeStruct((n_pad, _ADP), jnp.float32),
        ),
        mesh=_sc_mesh(),
        scratch_types=[
            pltpu.VMEM((_CH,), jnp.int32),
            pltpu.VMEM((_CH, _LD), jnp.float32),
            pltpu.VMEM((_CH, _ADP), jnp.float32),
            pltpu.SemaphoreType.DMA,
            pltpu.SemaphoreType.DMA,
        ],
    )
    def gather_k(lat_hbm, act_hbm, gidx_hbm, xg_hbm, ag_hbm,
                 idx_v, xrows_v, arows_v, sem_x, sem_a):
        wid = lax.axis_index("s") * 2 + lax.axis_index("c")
        for c in range(n_ch):
            base = wid * rows_per_w + c * _CH
            pltpu.sync_copy(gidx_hbm.at[pl.ds(base, _CH)], idx_v)
            pltpu.async_copy(lat_hbm.at[idx_v], xrows_v, sem_x).wait()
            pltpu.async_copy(act_hbm.at[idx_v], arows_v, sem_a).wait()
            pltpu.sync_copy(xrows_v, xg_hbm.at[pl.ds(base, _CH)])
            pltpu.sync_copy(arows_v, ag_hbm.at[pl.ds(base, _CH)])

    return gather_k(latents, actions_p, gidx)


def _gather_back(pad_out, slot, n):
    rows_per_w = n // _NW
    n_ch = rows_per_w // _CH

    @functools.partial(
        pl.kernel,
        out_type=jax.ShapeDtypeStruct((n, _LD), jnp.float32),
        mesh=_sc_mesh(),
        scratch_types=[
            pltpu.VMEM((_CH,), jnp.int32),
            pltpu.VMEM((_CH, _LD), jnp.float32),
            pltpu.SemaphoreType.DMA,
        ],
    )
    def back_k(pad_hbm, slot_hbm, out_hbm, idx_v, rows_v, sem):
        wid = lax.axis_index("s") * 2 + lax.axis_index("c")
        for c in range(n_ch):
            base = wid * rows_per_w + c * _CH
            pltpu.sync_copy(slot_hbm.at[pl.ds(base, _CH)], idx_v)
            pltpu.async_copy(pad_hbm.at[idx_v], rows_v, sem).wait()
            pltpu.sync_copy(rows_v, out_hbm.at[pl.ds(base, _CH)])

    return back_k(pad_out, slot)


def _mlp_body(bp_ref, x_ref, a_ref, w1z_ref, w1a_ref, b1_ref, w2_ref, b2_ref,
              o_ref):
    x = x_ref[...]
    a = a_ref[...]
    aterm = jnp.dot(a, w1a_ref[0], preferred_element_type=jnp.float32)
    for q in range(4):
        z = x[:, q * 192:(q + 1) * 192]
        h = jnp.dot(z, w1z_ref[0, q], preferred_element_type=jnp.float32)
        h = h + aterm[:, q * 128:(q + 1) * 128] + b1_ref[0, 0, q * 128:(q + 1) * 128]
        h = jnp.maximum(h, 0.0)
        y = jnp.dot(h, w2_ref[0, q], preferred_element_type=jnp.float32)
        o_ref[:, q * 192:(q + 1) * 192] = y + b2_ref[0, 0, q * 192:(q + 1) * 192]


def _mlp_blocks(xg, ag, bpol, W1z, W1a, b1f, W2p, b2f, n_pad, n_blk):
    grid_spec = pltpu.PrefetchScalarGridSpec(
        num_scalar_prefetch=1,
        grid=(n_blk,),
        in_specs=[
            pl.BlockSpec((_BLK, _LD), lambda k, bp: (k, 0)),
            pl.BlockSpec((_BLK, _ADP), lambda k, bp: (k, 0)),
            pl.BlockSpec((1, 4, 192, 128), lambda k, bp: (bp[k], 0, 0, 0)),
            pl.BlockSpec((1, _ADP, 512), lambda k, bp: (bp[k], 0, 0)),
            pl.BlockSpec((1, 1, 512), lambda k, bp: (bp[k], 0, 0)),
            pl.BlockSpec((1, 4, 128, 192), lambda k, bp: (bp[k], 0, 0, 0)),
            pl.BlockSpec((1, 1, 768), lambda k, bp: (bp[k], 0, 0)),
        ],
        out_specs=pl.BlockSpec((_BLK, _LD), lambda k, bp: (k, 0)),
    )
    return pl.pallas_call(
        _mlp_body,
        grid_spec=grid_spec,
        out_shape=jax.ShapeDtypeStruct((n_pad, _LD), jnp.float32),
    )(bpol, xg, ag, W1z, W1a, b1f, W2p, b2f)


def kernel(latents, policy_indices, actions, W1, b1, W2, b2):
    n = latents.shape[0]
    n_blk = n // _BLK + _P
    n_pad = n_blk * _BLK

    pol = policy_indices.astype(jnp.int32)
    slot, gidx, bpol = _routing_metadata(pol, n_pad, n_blk)
    W1z, W1a, b1f, W2p, b2f = _pack_weights(W1, b1, W2, b2)
    actions_p = jnp.zeros((n, _ADP), actions.dtype).at[:, :_AD].set(actions)

    xg, ag = _gather_sorted(latents, actions_p, gidx, n_pad)
    pad_out = _mlp_blocks(xg, ag, bpol, W1z, W1a, b1f, W2p, b2f, n_pad, n_blk)
    return _gather_back(pad_out, slot, n)


# speedup vs baseline: 2.3353x; 2.3353x over previous
import functools

import jax
import jax.numpy as jnp
from jax import lax
from jax.experimental import pallas as pl
from jax.experimental.pallas import tpu as pltpu
from jax.experimental.pallas import tpu_sc as plsc

_P = 8
_DP = 96
_AD = 8
_ADP = 128
_H = 64
_LD = _P * _DP
_BLK = 512
_NW = 32
_CH = 128


def _routing_metadata(pol, n_pad, n_blk):
    n = pol.shape[0]
    oh = (pol[:, None] == jnp.arange(_P, dtype=jnp.int32)[None, :]).astype(jnp.int32)
    rank = jnp.take_along_axis(jnp.cumsum(oh, axis=0) - oh, pol[:, None], axis=1)[:, 0]
    counts = jnp.sum(oh, axis=0)
    padded = ((counts + _BLK - 1) // _BLK) * _BLK
    seg_end = jnp.cumsum(padded)
    seg_start = seg_end - padded
    slot = seg_start[pol] + rank
    gidx = (jnp.arange(n_pad, dtype=jnp.int32) % n).at[slot].set(
        jnp.arange(n, dtype=jnp.int32))
    bpol = jnp.searchsorted(
        seg_end, jnp.arange(n_blk, dtype=jnp.int32) * _BLK, side="right")
    bpol = jnp.minimum(bpol, _P - 1).astype(jnp.int32)
    return slot, gidx, bpol


def _pack_weights(W1, b1, W2, b2):
    Wz = W1[:, :, :_DP, :]
    W1z = jnp.zeros((_P, 4, 2 * _DP, 2 * _H), W1.dtype)
    W1z = W1z.at[:, :, :_DP, :_H].set(Wz[:, 0::2])
    W1z = W1z.at[:, :, _DP:, _H:].set(Wz[:, 1::2])
    Wa = jnp.transpose(W1[:, :, _DP:, :], (0, 2, 1, 3)).reshape(_P, _AD, _P * _H)
    W1a = jnp.zeros((_P, _ADP, _P * _H), W1.dtype).at[:, :_AD, :].set(Wa)
    b1f = b1.reshape(_P, 1, _P * _H)
    W2p = jnp.zeros((_P, 4, 2 * _H, 2 * _DP), W2.dtype)
    W2p = W2p.at[:, :, :_H, :_DP].set(W2[:, 0::2])
    W2p = W2p.at[:, :, _H:, _DP:].set(W2[:, 1::2])
    b2f = b2.reshape(_P, 1, _P * _DP)
    return W1z, W1a, b1f, W2p, b2f


def _sc_mesh():
    return plsc.VectorSubcoreMesh(core_axis_name="c", subcore_axis_name="s")


def _gather_sorted(latents, actions_p, gidx, n_pad):
    rows_per_w = n_pad // _NW
    n_ch = rows_per_w // _CH

    @functools.partial(
        pl.kernel,
        out_type=(
            jax.ShapeDtypeStruct((n_pad, _LD), jnp.float32),
            jax.ShapeDtypeStruct((n_pad, _ADP), jnp.float32),
        ),
        mesh=_sc_mesh(),
        scratch_types=[
            pltpu.VMEM((_CH,), jnp.int32),
            pltpu.VMEM((_CH, _LD), jnp.float32),
            pltpu.VMEM((_CH, _ADP), jnp.float32),
            pltpu.SemaphoreType.DMA,
            pltpu.SemaphoreType.DMA,
        ],
    )
    def gather_k(lat_hbm, act_hbm, gidx_hbm, xg_hbm, ag_hbm,
                 idx_v, xrows_v, arows_v, sem_x, sem_a):
        wid = lax.axis_index("s") * 2 + lax.axis_index("c")
        for c in range(n_ch):
            base = wid * rows_per_w + c * _CH
            pltpu.sync_copy(gidx_hbm.at[pl.ds(base, _CH)], idx_v)
            cp_x = pltpu.async_copy(lat_hbm.at[idx_v], xrows_v, sem_x)
            cp_a = pltpu.async_copy(act_hbm.at[idx_v], arows_v, sem_a)
            cp_x.wait()
            cp_a.wait()
            pltpu.sync_copy(xrows_v, xg_hbm.at[pl.ds(base, _CH)])
            pltpu.sync_copy(arows_v, ag_hbm.at[pl.ds(base, _CH)])

    return gather_k(latents, actions_p, gidx)


def _gather_back(pad_out, slot, n):
    rows_per_w = n // _NW
    n_ch = rows_per_w // _CH

    @functools.partial(
        pl.kernel,
        out_type=jax.ShapeDtypeStruct((n, _LD), jnp.float32),
        mesh=_sc_mesh(),
        scratch_types=[
            pltpu.VMEM((_CH,), jnp.int32),
            pltpu.VMEM((_CH, _LD), jnp.float32),
            pltpu.SemaphoreType.DMA,
        ],
    )
    def back_k(pad_hbm, slot_hbm, out_hbm, idx_v, rows_v, sem):
        wid = lax.axis_index("s") * 2 + lax.axis_index("c")
        for c in range(n_ch):
            base = wid * rows_per_w + c * _CH
            pltpu.sync_copy(slot_hbm.at[pl.ds(base, _CH)], idx_v)
            pltpu.async_copy(pad_hbm.at[idx_v], rows_v, sem).wait()
            pltpu.sync_copy(rows_v, out_hbm.at[pl.ds(base, _CH)])

    return back_k(pad_out, slot)


def _mlp_body(bp_ref, x_ref, a_ref, w1z_ref, w1a_ref, b1_ref, w2_ref, b2_ref,
              o_ref):
    x = x_ref[...]
    a = a_ref[...]
    aterm = jnp.dot(a, w1a_ref[0], preferred_element_type=jnp.float32)
    for q in range(4):
        z = x[:, q * 192:(q + 1) * 192]
        h = jnp.dot(z, w1z_ref[0, q], preferred_element_type=jnp.float32)
        h = h + aterm[:, q * 128:(q + 1) * 128] + b1_ref[0, 0, q * 128:(q + 1) * 128]
        h = jnp.maximum(h, 0.0)
        y = jnp.dot(h, w2_ref[0, q], preferred_element_type=jnp.float32)
        o_ref[:, q * 192:(q + 1) * 192] = y + b2_ref[0, 0, q * 192:(q + 1) * 192]


def _mlp_blocks(xg, ag, bpol, W1z, W1a, b1f, W2p, b2f, n_pad, n_blk):
    grid_spec = pltpu.PrefetchScalarGridSpec(
        num_scalar_prefetch=1,
        grid=(n_blk,),
        in_specs=[
            pl.BlockSpec((_BLK, _LD), lambda k, bp: (k, 0)),
            pl.BlockSpec((_BLK, _ADP), lambda k, bp: (k, 0)),
            pl.BlockSpec((1, 4, 192, 128), lambda k, bp: (bp[k], 0, 0, 0)),
            pl.BlockSpec((1, _ADP, 512), lambda k, bp: (bp[k], 0, 0)),
            pl.BlockSpec((1, 1, 512), lambda k, bp: (bp[k], 0, 0)),
            pl.BlockSpec((1, 4, 128, 192), lambda k, bp: (bp[k], 0, 0, 0)),
            pl.BlockSpec((1, 1, 768), lambda k, bp: (bp[k], 0, 0)),
        ],
        out_specs=pl.BlockSpec((_BLK, _LD), lambda k, bp: (k, 0)),
    )
    return pl.pallas_call(
        _mlp_body,
        grid_spec=grid_spec,
        out_shape=jax.ShapeDtypeStruct((n_pad, _LD), jnp.float32),
    )(bpol, xg, ag, W1z, W1a, b1f, W2p, b2f)


def kernel(latents, policy_indices, actions, W1, b1, W2, b2):
    n = latents.shape[0]
    n_blk = n // _BLK + _P
    n_pad = n_blk * _BLK

    pol = policy_indices.astype(jnp.int32)
    slot, gidx, bpol = _routing_metadata(pol, n_pad, n_blk)
    W1z, W1a, b1f, W2p, b2f = _pack_weights(W1, b1, W2, b2)
    actions_p = jnp.zeros((n, _ADP), actions.dtype).at[:, :_AD].set(actions)

    xg, ag = _gather_sorted(latents, actions_p, gidx, n_pad)
    pad_out = _mlp_blocks(xg, ag, bpol, W1z, W1a, b1f, W2p, b2f, n_pad, n_blk)
    return _gather_back(pad_out, slot, n)


# speedup vs baseline: 2.8571x; 1.2235x over previous
import functools

import jax
import jax.numpy as jnp
from jax import lax
from jax.experimental import pallas as pl
from jax.experimental.pallas import tpu as pltpu
from jax.experimental.pallas import tpu_sc as plsc

_P = 8
_DP = 96
_AD = 8
_ADP = 128
_H = 64
_LD = _P * _DP
_XW = _LD + _ADP
_BLK = 128
_NW = 32
_TPW = 256
_SCH = 64


def _sc_mesh():
    return plsc.VectorSubcoreMesh(core_axis_name="c", subcore_axis_name="s")


def _wid():
    return lax.axis_index("s") * 2 + lax.axis_index("c")


def _hist_ranks(pol):
    @functools.partial(
        pl.kernel,
        out_type=(
            jax.ShapeDtypeStruct((_NW, 16), jnp.int32),
            jax.ShapeDtypeStruct((_NW * 2, 128), jnp.int32),
        ),
        mesh=_sc_mesh(),
        scratch_types=[
            pltpu.VMEM((_TPW,), jnp.int32),
            pltpu.VMEM((2, 128), jnp.int32),
            pltpu.VMEM((16,), jnp.int32),
        ],
        compiler_params=pltpu.CompilerParams(needs_layout_passes=False),
    )
    def m1_k(pol_hbm, hist_hbm, rank_hbm, polv, rnk, histv):
        wid = _wid()
        pltpu.sync_copy(pol_hbm.at[pl.ds(wid * _TPW, _TPW)], polv)
        lanes = lax.iota(jnp.int32, 16)
        hist = jnp.zeros((16,), jnp.int32)
        for i in range(_P):
            run = jnp.zeros((16,), jnp.int32)
            for v in range(16):
                xv = polv[pl.ds(v * 16, 16)]
                m = xv == i
                mi = m.astype(jnp.int32)
                pre = plsc.cumsum(mi) - mi + run
                old = rnk[v // 8, pl.ds((v % 8) * 16, 16)]
                rnk[v // 8, pl.ds((v % 8) * 16, 16)] = jnp.where(m, pre, old)
                run = run + plsc.all_reduce_population_count(m)
            hist = jnp.where(lanes == i, run, hist)
        histv[...] = hist
        pltpu.sync_copy(histv, hist_hbm.at[wid])
        pltpu.sync_copy(rnk, rank_hbm.at[pl.ds(wid * 2, 2)])

    return m1_k(pol)


def _route_scatter(latents, actions_p, pol, hist, rank, n_pad, n_blk):
    @functools.partial(
        pl.kernel,
        out_type=(
            jax.ShapeDtypeStruct((n_pad, _XW), jnp.float32),
            jax.ShapeDtypeStruct((_NW * 4, _SCH), jnp.int32),
            jax.ShapeDtypeStruct((80,), jnp.int32),
        ),
        mesh=_sc_mesh(),
        scratch_types=[
            pltpu.VMEM((_TPW,), jnp.int32),
            pltpu.VMEM((2, 128), jnp.int32),
            pltpu.VMEM((_NW, 16), jnp.int32),
            pltpu.VMEM((16,), jnp.int32),
            pltpu.VMEM((80,), jnp.int32),
            pltpu.VMEM((4, _SCH), jnp.int32),
            pltpu.VMEM((_SCH, _XW), jnp.float32),
            pltpu.VMEM((_SCH, _XW), jnp.float32),
            pltpu.SemaphoreType.DMA,
            pltpu.SemaphoreType.DMA,
        ],
        compiler_params=pltpu.CompilerParams(needs_layout_passes=False),
    )
    def a_k(lat_hbm, act_hbm, pol_hbm, hist_hbm, rank_hbm,
            xgc_hbm, slot_hbm, spol_hbm,
            polv, rankv, histv, basev, spolv, slot_idx,
            rows0, rows1, sem0, sem1):
        wid = _wid()
        pltpu.sync_copy(pol_hbm.at[pl.ds(wid * _TPW, _TPW)], polv)
        pltpu.sync_copy(rank_hbm.at[pl.ds(wid * 2, 2)], rankv)
        pltpu.sync_copy(hist_hbm, histv)
        widv = jnp.full((16,), wid, jnp.int32)
        cnt = jnp.zeros((16,), jnp.int32)
        pre = jnp.zeros((16,), jnp.int32)
        zero16 = jnp.zeros((16,), jnp.int32)
        for t in range(_NW):
            h = histv[t]
            cnt = cnt + h
            pre = pre + jnp.where(jnp.full((16,), t, jnp.int32) < widv, h, zero16)
        padded = ((cnt + (_BLK - 1)) >> 7) << 7
        seg_end = plsc.cumsum(padded)
        base = seg_end - padded + pre
        basev[...] = base
        for v in range(16):
            p = polv[pl.ds(v * 16, 16)]
            b = plsc.load_gather(basev, [p])
            r = rankv[v // 8, pl.ds((v % 8) * 16, 16)]
            slot_idx[v // 4, pl.ds((v % 4) * 16, 16)] = b + r
        pltpu.sync_copy(slot_idx, slot_hbm.at[pl.ds(wid * 4, 4)])

        @pl.when(wid == 0)
        def _spol():
            lanes = lax.iota(jnp.int32, 16)
            zero16 = jnp.zeros((16,), jnp.int32)
            for g in range(5):
                bstart = (lanes + g * 16) * _BLK
                acc = jnp.zeros((16,), jnp.int32)
                for i in range(_P):
                    se_i = jnp.sum(jnp.where(lanes == i, seg_end, zero16))
                    acc = acc + (bstart >= se_i).astype(jnp.int32)
                spolv[pl.ds(g * 16, 16)] = jnp.minimum(acc, _P - 1)
            pltpu.sync_copy(spolv, spol_hbm)

        bufs = (rows0, rows1)
        sems = (sem0, sem1)
        cps = [None, None]
        for c in range(4):
            b = c % 2
            if cps[b] is not None:
                cps[b].wait()
            tb = wid * _TPW + c * _SCH
            pltpu.sync_copy(lat_hbm.at[pl.ds(tb, _SCH)],
                            bufs[b].at[:, pl.ds(0, _LD)])
            pltpu.sync_copy(act_hbm.at[pl.ds(tb, _SCH)],
                            bufs[b].at[:, pl.ds(_LD, _ADP)])
            cps[b] = pltpu.async_copy(bufs[b], xgc_hbm.at[slot_idx.at[c]],
                                      sems[b])
        cps[0].wait()
        cps[1].wait()

    return a_k(latents, actions_p, pol, hist, rank)


def _gather_back(pad_out, slot, n):
    @functools.partial(
        pl.kernel,
        out_type=jax.ShapeDtypeStruct((n, _LD), jnp.float32),
        mesh=_sc_mesh(),
        scratch_types=[
            pltpu.VMEM((4, _SCH), jnp.int32),
            pltpu.VMEM((_SCH, _LD), jnp.float32),
            pltpu.VMEM((_SCH, _LD), jnp.float32),
            pltpu.SemaphoreType.DMA,
            pltpu.SemaphoreType.DMA,
        ],
    )
    def b_k(pad_hbm, slot_hbm, out_hbm, sidx, rows0, rows1, sem0, sem1):
        wid = _wid()
        pltpu.sync_copy(slot_hbm.at[pl.ds(wid * 4, 4)], sidx)
        bufs = (rows0, rows1)
        sems = (sem0, sem1)
        cps = [None, None]
        for c in range(4):
            b = c % 2
            if cps[b] is not None:
                cps[b].wait()
                pltpu.sync_copy(bufs[b],
                                out_hbm.at[pl.ds(wid * _TPW + (c - 2) * _SCH,
                                                 _SCH)])
            cps[b] = pltpu.async_copy(pad_hbm.at[sidx.at[c]], bufs[b], sems[b])
        for c in range(2):
            cps[c].wait()
            pltpu.sync_copy(bufs[c],
                            out_hbm.at[pl.ds(wid * _TPW + (c + 2) * _SCH,
                                             _SCH)])

    return b_k(pad_out, slot)


def _mlp_body(sp_ref, x_ref, w1_ref, w1a_ref, w2_ref, b2_ref, o_ref):
    x = x_ref[...]
    aterm = jnp.dot(x[:, _LD:], w1a_ref[0], preferred_element_type=jnp.float32)
    for j in range(_P):
        z = x[:, _DP * j:_DP * (j + 1)]
        h = jnp.dot(z, w1_ref[0, j, 0:_DP, :],
                    preferred_element_type=jnp.float32)
        h = jnp.maximum(h + aterm[:, _H * j:_H * (j + 1)], 0.0)
        y = jnp.dot(h, w2_ref[0, j], preferred_element_type=jnp.float32)
        o_ref[:, _DP * j:_DP * (j + 1)] = y + b2_ref[0, j]


def _mlp_blocks(xgc, spol, W1, W1a, W2, b2, n_pad, n_blk):
    grid_spec = pltpu.PrefetchScalarGridSpec(
        num_scalar_prefetch=1,
        grid=(n_blk,),
        in_specs=[
            pl.BlockSpec((_BLK, _XW), lambda k, sp: (k, 0)),
            pl.BlockSpec((1, _P, _DP + _AD, _H), lambda k, sp: (sp[k], 0, 0, 0)),
            pl.BlockSpec((1, _ADP, _P * _H), lambda k, sp: (sp[k], 0, 0)),
            pl.BlockSpec((1, _P, _H, _DP), lambda k, sp: (sp[k], 0, 0, 0)),
            pl.BlockSpec((1, _P, _DP), lambda k, sp: (sp[k], 0, 0)),
        ],
        out_specs=pl.BlockSpec((_BLK, _LD), lambda k, sp: (k, 0)),
    )
    return pl.pallas_call(
        _mlp_body,
        grid_spec=grid_spec,
        out_shape=jax.ShapeDtypeStruct((n_pad, _LD), jnp.float32),
    )(spol, xgc, W1, W1a, W2, b2)


def kernel(latents, policy_indices, actions, W1, b1, W2, b2):
    n = latents.shape[0]
    n_pad = n + _P * _BLK
    n_blk = n_pad // _BLK

    pol = policy_indices.astype(jnp.int32)
    actions_p = (jnp.zeros((n, _ADP), actions.dtype)
                 .at[:, :_AD].set(actions)
                 .at[:, _AD].set(1.0))
    Wa = jnp.transpose(W1[:, :, _DP:, :], (0, 2, 1, 3)).reshape(_P, _AD, _P * _H)
    W1a = (jnp.zeros((_P, _ADP, _P * _H), W1.dtype)
           .at[:, :_AD].set(Wa)
           .at[:, _AD].set(b1.reshape(_P, _P * _H)))

    hist, rank = _hist_ranks(pol)
    xgc, slot, spol = _route_scatter(latents, actions_p, pol, hist, rank,
                                     n_pad, n_blk)
    pad_out = _mlp_blocks(xgc, spol, W1, W1a, W2, b2, n_pad, n_blk)
    return _gather_back(pad_out, slot, n)


# speedup vs baseline: 2.8910x; 1.0119x over previous
import functools

import jax
import jax.numpy as jnp
from jax import lax
from jax.experimental import pallas as pl
from jax.experimental.pallas import tpu as pltpu
from jax.experimental.pallas import tpu_sc as plsc

_P = 8
_DP = 96
_AD = 8
_ADP = 128
_H = 64
_LD = _P * _DP
_XW = _LD + _ADP
_BLK = 128
_NW = 32
_TPW = 256
_SCH = 64


def _sc_mesh():
    return plsc.VectorSubcoreMesh(core_axis_name="c", subcore_axis_name="s")


def _wid():
    return lax.axis_index("s") * 2 + lax.axis_index("c")


def _hist_ranks(pol):
    @functools.partial(
        pl.kernel,
        out_type=(
            jax.ShapeDtypeStruct((_NW, 16), jnp.int32),
            jax.ShapeDtypeStruct((_NW * 2, 128), jnp.int32),
        ),
        mesh=_sc_mesh(),
        scratch_types=[
            pltpu.VMEM((_TPW,), jnp.int32),
            pltpu.VMEM((2, 128), jnp.int32),
            pltpu.VMEM((16,), jnp.int32),
        ],
        compiler_params=pltpu.CompilerParams(needs_layout_passes=False),
    )
    def m1_k(pol_hbm, hist_hbm, rank_hbm, polv, rnk, histv):
        wid = _wid()
        pltpu.sync_copy(pol_hbm.at[pl.ds(wid * _TPW, _TPW)], polv)
        lanes = lax.iota(jnp.int32, 16)
        hist = jnp.zeros((16,), jnp.int32)
        for i in range(_P):
            run = jnp.zeros((16,), jnp.int32)
            for v in range(16):
                xv = polv[pl.ds(v * 16, 16)]
                m = xv == i
                mi = m.astype(jnp.int32)
                pre = plsc.cumsum(mi) - mi + run
                old = rnk[v // 8, pl.ds((v % 8) * 16, 16)]
                rnk[v // 8, pl.ds((v % 8) * 16, 16)] = jnp.where(m, pre, old)
                run = run + plsc.all_reduce_population_count(m)
            hist = jnp.where(lanes == i, run, hist)
        histv[...] = hist
        pltpu.sync_copy(histv, hist_hbm.at[wid])
        pltpu.sync_copy(rnk, rank_hbm.at[pl.ds(wid * 2, 2)])

    return m1_k(pol)


def _route_scatter(latents, actions_p, pol, hist, rank, n_pad, n_blk):
    @functools.partial(
        pl.kernel,
        out_type=(
            jax.ShapeDtypeStruct((n_pad, _XW), jnp.float32),
            jax.ShapeDtypeStruct((_NW * 4, _SCH), jnp.int32),
            jax.ShapeDtypeStruct((80,), jnp.int32),
        ),
        mesh=_sc_mesh(),
        scratch_types=[
            pltpu.VMEM((_TPW,), jnp.int32),
            pltpu.VMEM((2, 128), jnp.int32),
            pltpu.VMEM((_NW, 16), jnp.int32),
            pltpu.VMEM((16,), jnp.int32),
            pltpu.VMEM((80,), jnp.int32),
            pltpu.VMEM((4, _SCH), jnp.int32),
            pltpu.VMEM((_SCH, _XW), jnp.float32),
            pltpu.VMEM((_SCH, _XW), jnp.float32),
            pltpu.SemaphoreType.DMA,
            pltpu.SemaphoreType.DMA,
        ],
        compiler_params=pltpu.CompilerParams(needs_layout_passes=False),
    )
    def a_k(lat_hbm, act_hbm, pol_hbm, hist_hbm, rank_hbm,
            xgc_hbm, slot_hbm, spol_hbm,
            polv, rankv, histv, basev, spolv, slot_idx,
            rows0, rows1, sem0, sem1):
        wid = _wid()
        pltpu.sync_copy(pol_hbm.at[pl.ds(wid * _TPW, _TPW)], polv)
        pltpu.sync_copy(rank_hbm.at[pl.ds(wid * 2, 2)], rankv)
        pltpu.sync_copy(hist_hbm, histv)
        widv = jnp.full((16,), wid, jnp.int32)
        cnt = jnp.zeros((16,), jnp.int32)
        pre = jnp.zeros((16,), jnp.int32)
        zero16 = jnp.zeros((16,), jnp.int32)
        for t in range(_NW):
            h = histv[t]
            cnt = cnt + h
            pre = pre + jnp.where(jnp.full((16,), t, jnp.int32) < widv, h, zero16)
        padded = ((cnt + (_BLK - 1)) >> 7) << 7
        seg_end = plsc.cumsum(padded)
        base = seg_end - padded + pre
        basev[...] = base
        for v in range(16):
            p = polv[pl.ds(v * 16, 16)]
            b = plsc.load_gather(basev, [p])
            r = rankv[v // 8, pl.ds((v % 8) * 16, 16)]
            slot_idx[v // 4, pl.ds((v % 4) * 16, 16)] = b + r
        pltpu.sync_copy(slot_idx, slot_hbm.at[pl.ds(wid * 4, 4)])

        @pl.when(wid == 0)
        def _spol():
            lanes = lax.iota(jnp.int32, 16)
            zero16 = jnp.zeros((16,), jnp.int32)
            for g in range(5):
                bstart = (lanes + g * 16) * _BLK
                acc = jnp.zeros((16,), jnp.int32)
                for i in range(_P):
                    se_i = jnp.sum(jnp.where(lanes == i, seg_end, zero16))
                    acc = acc + (bstart >= se_i).astype(jnp.int32)
                spolv[pl.ds(g * 16, 16)] = jnp.minimum(acc, _P - 1)
            pltpu.sync_copy(spolv, spol_hbm)

        bufs = (rows0, rows1)
        sems = (sem0, sem1)
        cps = [None, None]
        for c in range(4):
            b = c % 2
            if cps[b] is not None:
                cps[b].wait()
            tb = wid * _TPW + c * _SCH
            pltpu.sync_copy(lat_hbm.at[pl.ds(tb, _SCH)],
                            bufs[b].at[:, pl.ds(0, _LD)])
            pltpu.sync_copy(act_hbm.at[pl.ds(tb, _SCH)],
                            bufs[b].at[:, pl.ds(_LD, _ADP)])
            cps[b] = pltpu.async_copy(bufs[b], xgc_hbm.at[slot_idx.at[c]],
                                      sems[b])
        cps[0].wait()
        cps[1].wait()

    return a_k(latents, actions_p, pol, hist, rank)


def _gather_back(pad_out, slot, n):
    @functools.partial(
        pl.kernel,
        out_type=jax.ShapeDtypeStruct((n, _LD), jnp.float32),
        mesh=_sc_mesh(),
        scratch_types=[
            pltpu.VMEM((4, _SCH), jnp.int32),
            pltpu.VMEM((_SCH, _LD), jnp.float32),
            pltpu.VMEM((_SCH, _LD), jnp.float32),
            pltpu.SemaphoreType.DMA,
            pltpu.SemaphoreType.DMA,
        ],
    )
    def b_k(pad_hbm, slot_hbm, out_hbm, sidx, rows0, rows1, sem0, sem1):
        wid = _wid()
        pltpu.sync_copy(slot_hbm.at[pl.ds(wid * 4, 4)], sidx)
        bufs = (rows0, rows1)
        sems = (sem0, sem1)
        cps = [None, None]
        for c in range(4):
            b = c % 2
            if cps[b] is not None:
                cps[b].wait()
                pltpu.sync_copy(bufs[b],
                                out_hbm.at[pl.ds(wid * _TPW + (c - 2) * _SCH,
                                                 _SCH)])
            cps[b] = pltpu.async_copy(pad_hbm.at[sidx.at[c]], bufs[b], sems[b])
        for c in range(2):
            cps[c].wait()
            pltpu.sync_copy(bufs[c],
                            out_hbm.at[pl.ds(wid * _TPW + (c + 2) * _SCH,
                                             _SCH)])

    return b_k(pad_out, slot)


def _mlp_body(sp_ref, x_ref, w1_ref, w1a_ref, w2_ref, b2_ref, o_ref):
    x = x_ref[...].astype(jnp.bfloat16)
    aterm = jnp.dot(x[:, _LD:], w1a_ref[0], preferred_element_type=jnp.float32)
    for j in range(_P):
        z = x[:, _DP * j:_DP * (j + 1)]
        h = jnp.dot(z, w1_ref[0, j, 0:_DP, :],
                    preferred_element_type=jnp.float32)
        h = jnp.maximum(h + aterm[:, _H * j:_H * (j + 1)], 0.0)
        y = jnp.dot(h.astype(jnp.bfloat16), w2_ref[0, j],
                    preferred_element_type=jnp.float32)
        o_ref[:, _DP * j:_DP * (j + 1)] = y + b2_ref[0, j]


def _mlp_blocks(xgc, spol, W1, W1a, W2, b2, n_pad, n_blk):
    grid_spec = pltpu.PrefetchScalarGridSpec(
        num_scalar_prefetch=1,
        grid=(n_blk,),
        in_specs=[
            pl.BlockSpec((_BLK, _XW), lambda k, sp: (k, 0)),
            pl.BlockSpec((1, _P, _DP + _AD, _H), lambda k, sp: (sp[k], 0, 0, 0)),
            pl.BlockSpec((1, _ADP, _P * _H), lambda k, sp: (sp[k], 0, 0)),
            pl.BlockSpec((1, _P, _H, _DP), lambda k, sp: (sp[k], 0, 0, 0)),
            pl.BlockSpec((1, _P, _DP), lambda k, sp: (sp[k], 0, 0)),
        ],
        out_specs=pl.BlockSpec((_BLK, _LD), lambda k, sp: (k, 0)),
    )
    return pl.pallas_call(
        _mlp_body,
        grid_spec=grid_spec,
        out_shape=jax.ShapeDtypeStruct((n_pad, _LD), jnp.float32),
    )(spol, xgc, W1, W1a, W2, b2)


def kernel(latents, policy_indices, actions, W1, b1, W2, b2):
    n = latents.shape[0]
    n_pad = n + _P * _BLK
    n_blk = n_pad // _BLK

    pol = policy_indices.astype(jnp.int32)
    actions_p = (jnp.zeros((n, _ADP), actions.dtype)
                 .at[:, :_AD].set(actions)
                 .at[:, _AD].set(1.0))
    Wa = jnp.transpose(W1[:, :, _DP:, :], (0, 2, 1, 3)).reshape(_P, _AD, _P * _H)
    W1a = (jnp.zeros((_P, _ADP, _P * _H), W1.dtype)
           .at[:, :_AD].set(Wa)
           .at[:, _AD].set(b1.reshape(_P, _P * _H)))

    hist, rank = _hist_ranks(pol)
    xgc, slot, spol = _route_scatter(latents, actions_p, pol, hist, rank,
                                     n_pad, n_blk)
    pad_out = _mlp_blocks(xgc, spol, W1.astype(jnp.bfloat16),
                          W1a.astype(jnp.bfloat16), W2.astype(jnp.bfloat16),
                          b2, n_pad, n_blk)
    return _gather_back(pad_out, slot, n)


# speedup vs baseline: 3.5994x; 1.2450x over previous
import functools

import jax
import jax.numpy as jnp
from jax import lax
from jax.experimental import pallas as pl
from jax.experimental.pallas import tpu as pltpu
from jax.experimental.pallas import tpu_sc as plsc

_P = 8
_DP = 96
_AD = 8
_ADP = 128
_H = 64
_LD = _P * _DP
_XW = _LD + _ADP
_BLK = 128
_NW = 32
_TPW = 256
_SCH = 64


def _sc_mesh():
    return plsc.VectorSubcoreMesh(core_axis_name="c", subcore_axis_name="s")


def _wid():
    return lax.axis_index("s") * 2 + lax.axis_index("c")


def _hist_ranks(pol):
    @functools.partial(
        pl.kernel,
        out_type=(
            jax.ShapeDtypeStruct((_NW, 16), jnp.int32),
            jax.ShapeDtypeStruct((_NW * 2, 128), jnp.int32),
        ),
        mesh=_sc_mesh(),
        scratch_types=[
            pltpu.VMEM((_TPW,), jnp.int32),
            pltpu.VMEM((2, 128), jnp.int32),
            pltpu.VMEM((16,), jnp.int32),
        ],
        compiler_params=pltpu.CompilerParams(needs_layout_passes=False),
    )
    def m1_k(pol_hbm, hist_hbm, rank_hbm, polv, rnk, histv):
        wid = _wid()
        pltpu.sync_copy(pol_hbm.at[pl.ds(wid * _TPW, _TPW)], polv)
        lanes = lax.iota(jnp.int32, 16)
        hist = jnp.zeros((16,), jnp.int32)
        for i in range(_P):
            run = jnp.zeros((16,), jnp.int32)
            for v in range(16):
                xv = polv[pl.ds(v * 16, 16)]
                m = xv == i
                mi = m.astype(jnp.int32)
                pre = plsc.cumsum(mi) - mi + run
                old = rnk[v // 8, pl.ds((v % 8) * 16, 16)]
                rnk[v // 8, pl.ds((v % 8) * 16, 16)] = jnp.where(m, pre, old)
                run = run + plsc.all_reduce_population_count(m)
            hist = jnp.where(lanes == i, run, hist)
        histv[...] = hist
        pltpu.sync_copy(histv, hist_hbm.at[wid])
        pltpu.sync_copy(rnk, rank_hbm.at[pl.ds(wid * 2, 2)])

    return m1_k(pol)


def _route_scatter(latents, actions_p, pol, hist, rank, n_pad, n_blk):
    @functools.partial(
        pl.kernel,
        out_type=(
            jax.ShapeDtypeStruct((n_pad, _XW), jnp.float32),
            jax.ShapeDtypeStruct((_NW * 4, _SCH), jnp.int32),
            jax.ShapeDtypeStruct((80,), jnp.int32),
        ),
        mesh=_sc_mesh(),
        scratch_types=[
            pltpu.VMEM((_TPW,), jnp.int32),
            pltpu.VMEM((2, 128), jnp.int32),
            pltpu.VMEM((_NW, 16), jnp.int32),
            pltpu.VMEM((16,), jnp.int32),
            pltpu.VMEM((80,), jnp.int32),
            pltpu.VMEM((4, _SCH), jnp.int32),
            pltpu.VMEM((_SCH, _XW), jnp.float32),
            pltpu.VMEM((_SCH, _XW), jnp.float32),
            pltpu.SemaphoreType.DMA,
            pltpu.SemaphoreType.DMA,
        ],
        compiler_params=pltpu.CompilerParams(needs_layout_passes=False),
    )
    def a_k(lat_hbm, act_hbm, pol_hbm, hist_hbm, rank_hbm,
            xgc_hbm, slot_hbm, spol_hbm,
            polv, rankv, histv, basev, spolv, slot_idx,
            rows0, rows1, sem0, sem1):
        wid = _wid()
        pltpu.sync_copy(pol_hbm.at[pl.ds(wid * _TPW, _TPW)], polv)
        pltpu.sync_copy(rank_hbm.at[pl.ds(wid * 2, 2)], rankv)
        pltpu.sync_copy(hist_hbm, histv)
        widv = jnp.full((16,), wid, jnp.int32)
        cnt = jnp.zeros((16,), jnp.int32)
        pre = jnp.zeros((16,), jnp.int32)
        zero16 = jnp.zeros((16,), jnp.int32)
        for t in range(_NW):
            h = histv[t]
            cnt = cnt + h
            pre = pre + jnp.where(jnp.full((16,), t, jnp.int32) < widv, h, zero16)
        padded = ((cnt + (_BLK - 1)) >> 7) << 7
        seg_end = plsc.cumsum(padded)
        base = seg_end - padded + pre
        basev[...] = base
        for v in range(16):
            p = polv[pl.ds(v * 16, 16)]
            b = plsc.load_gather(basev, [p])
            r = rankv[v // 8, pl.ds((v % 8) * 16, 16)]
            slot_idx[v // 4, pl.ds((v % 4) * 16, 16)] = b + r
        pltpu.sync_copy(slot_idx, slot_hbm.at[pl.ds(wid * 4, 4)])

        @pl.when(wid == 0)
        def _spol():
            lanes = lax.iota(jnp.int32, 16)
            zero16 = jnp.zeros((16,), jnp.int32)
            for g in range(5):
                bstart = (lanes + g * 16) * _BLK
                acc = jnp.zeros((16,), jnp.int32)
                for i in range(_P):
                    se_i = jnp.sum(jnp.where(lanes == i, seg_end, zero16))
                    acc = acc + (bstart >= se_i).astype(jnp.int32)
                spolv[pl.ds(g * 16, 16)] = jnp.minimum(acc, _P - 1)
            pltpu.sync_copy(spolv, spol_hbm)

        bufs = (rows0, rows1)
        sems = (sem0, sem1)
        cps = [None, None]
        for c in range(4):
            b = c % 2
            if cps[b] is not None:
                cps[b].wait()
            tb = wid * _TPW + c * _SCH
            pltpu.sync_copy(lat_hbm.at[pl.ds(tb, _SCH)],
                            bufs[b].at[:, pl.ds(0, _LD)])
            pltpu.sync_copy(act_hbm.at[pl.ds(tb, _SCH)],
                            bufs[b].at[:, pl.ds(_LD, _ADP)])
            cps[b] = pltpu.async_copy(bufs[b], xgc_hbm.at[slot_idx.at[c]],
                                      sems[b])
        cps[0].wait()
        cps[1].wait()

    return a_k(latents, actions_p, pol, hist, rank)


def _gather_back(pad_out, slot, n):
    @functools.partial(
        pl.kernel,
        out_type=jax.ShapeDtypeStruct((n, _LD), jnp.float32),
        mesh=_sc_mesh(),
        scratch_types=[
            pltpu.VMEM((4, _SCH), jnp.int32),
            pltpu.VMEM((_SCH, _LD), jnp.float32),
            pltpu.VMEM((_SCH, _LD), jnp.float32),
            pltpu.SemaphoreType.DMA,
            pltpu.SemaphoreType.DMA,
        ],
    )
    def b_k(pad_hbm, slot_hbm, out_hbm, sidx, rows0, rows1, sem0, sem1):
        wid = _wid()
        pltpu.sync_copy(slot_hbm.at[pl.ds(wid * 4, 4)], sidx)
        bufs = (rows0, rows1)
        sems = (sem0, sem1)
        cps = [None, None]
        for c in range(4):
            b = c % 2
            if cps[b] is not None:
                cps[b].wait()
                pltpu.sync_copy(bufs[b],
                                out_hbm.at[pl.ds(wid * _TPW + (c - 2) * _SCH,
                                                 _SCH)])
            cps[b] = pltpu.async_copy(pad_hbm.at[sidx.at[c]], bufs[b], sems[b])
        for c in range(2):
            cps[c].wait()
            pltpu.sync_copy(bufs[c],
                            out_hbm.at[pl.ds(wid * _TPW + (c + 2) * _SCH,
                                             _SCH)])

    return b_k(pad_out, slot)


def _mlp_body(sp_ref, x_ref, w1_ref, w1a_ref, w2_ref, b2_ref, o_ref):
    k = pl.program_id(0)
    for q in range(4):
        p = sp_ref[k * 4 + q]
        x = x_ref[q * _BLK:(q + 1) * _BLK, :].astype(jnp.bfloat16)
        aterm = jnp.dot(x[:, _LD:], w1a_ref[p],
                        preferred_element_type=jnp.float32)
        for j in range(_P):
            z = x[:, _DP * j:_DP * (j + 1)]
            h = jnp.dot(z, w1_ref[p, j, 0:_DP, :],
                        preferred_element_type=jnp.float32)
            h = jnp.maximum(h + aterm[:, _H * j:_H * (j + 1)], 0.0)
            y = jnp.dot(h.astype(jnp.bfloat16), w2_ref[p, j],
                        preferred_element_type=jnp.float32)
            o_ref[q * _BLK:(q + 1) * _BLK, _DP * j:_DP * (j + 1)] = (
                y + b2_ref[p, j])


def _mlp_blocks(xgc, spol, W1, W1a, W2, b2, n_pad, n_blk):
    grid_spec = pltpu.PrefetchScalarGridSpec(
        num_scalar_prefetch=1,
        grid=(n_blk // 4,),
        in_specs=[
            pl.BlockSpec((4 * _BLK, _XW), lambda k, sp: (k, 0)),
            pl.BlockSpec((_P, _P, _DP + _AD, _H), lambda k, sp: (0, 0, 0, 0)),
            pl.BlockSpec((_P, _ADP, _P * _H), lambda k, sp: (0, 0, 0)),
            pl.BlockSpec((_P, _P, _H, _DP), lambda k, sp: (0, 0, 0, 0)),
            pl.BlockSpec((_P, _P, _DP), lambda k, sp: (0, 0, 0)),
        ],
        out_specs=pl.BlockSpec((4 * _BLK, _LD), lambda k, sp: (k, 0)),
    )
    return pl.pallas_call(
        _mlp_body,
        grid_spec=grid_spec,
        out_shape=jax.ShapeDtypeStruct((n_pad, _LD), jnp.float32),
    )(spol, xgc, W1, W1a, W2, b2)


def kernel(latents, policy_indices, actions, W1, b1, W2, b2):
    n = latents.shape[0]
    n_pad = n + _P * _BLK
    n_blk = n_pad // _BLK

    pol = policy_indices.astype(jnp.int32)
    actions_p = (jnp.zeros((n, _ADP), actions.dtype)
                 .at[:, :_AD].set(actions)
                 .at[:, _AD].set(1.0))
    Wa = jnp.transpose(W1[:, :, _DP:, :], (0, 2, 1, 3)).reshape(_P, _AD, _P * _H)
    W1a = (jnp.zeros((_P, _ADP, _P * _H), W1.dtype)
           .at[:, :_AD].set(Wa)
           .at[:, _AD].set(b1.reshape(_P, _P * _H)))

    hist, rank = _hist_ranks(pol)
    xgc, slot, spol = _route_scatter(latents, actions_p, pol, hist, rank,
                                     n_pad, n_blk)
    pad_out = _mlp_blocks(xgc, spol, W1.astype(jnp.bfloat16),
                          W1a.astype(jnp.bfloat16), W2.astype(jnp.bfloat16),
                          b2, n_pad, n_blk)
    return _gather_back(pad_out, slot, n)
